# baseline (speedup 1.0000x reference)
import functools

import jax
import jax.numpy as jnp
from jax.experimental import pallas as pl
from jax.experimental.pallas import tpu as pltpu

_GNN_ITER = 3
_TILE1 = 1024
_TILE2 = 1024


def _mm(a, b):
    return jax.lax.dot_general(a, b, (((1,), (0,)), ((), ())),
                               preferred_element_type=jnp.float32)


def _mlp_update(h_t, m, Wm1h_ref, Wm1m_ref, bm1_ref, Wm2_ref, bm2_ref,
                Wm3_ref, bm3_ref):
    u = jnp.maximum(_mm(h_t, Wm1h_ref[...]) + _mm(m, Wm1m_ref[...])
                    + bm1_ref[...], 0.0)
    u = jnp.maximum(_mm(u, Wm2_ref[...]) + bm2_ref[...], 0.0)
    u = _mm(u, Wm3_ref[...]) + bm3_ref[...]
    return h_t + u


def _pass1(x_ref, adj_ref, W_in_ref, b_in_ref, Wm1h_ref, Wm1m_ref, bm1_ref,
           Wm2_ref, bm2_ref, Wm3_ref, bm3_ref, h1_ref, p_ref, h0,
           *, n_b, tile, n_nodes):
    b = pl.program_id(0)
    i = pl.program_id(1)

    @pl.when((b == 0) & (i == 0))
    def _prologue():
        for bb in range(n_b):
            h0[bb] = jnp.tanh(_mm(x_ref[bb], W_in_ref[...]) + b_in_ref[...])

    adj_t = adj_ref[0]
    deg = jnp.sum(adj_t, axis=1, keepdims=True)
    rdeg = 1.0 / jnp.maximum(deg, 1.0)
    p_ref[0] = (adj_t * (float(n_nodes) * rdeg)).astype(jnp.float8_e4m3fn)
    m = _mm(adj_t, h0[b]) * rdeg
    h_t = h0[b, pl.ds(i * tile, tile)]
    h1_ref[0] = _mlp_update(h_t, m, Wm1h_ref, Wm1m_ref, bm1_ref,
                            Wm2_ref, bm2_ref, Wm3_ref, bm3_ref)


def _pass2(p_ref, h1_ref, Wm1h_ref, Wm1m_ref, bm1_ref, Wm2_ref, bm2_ref,
           Wm3_ref, bm3_ref, Wr1m_ref, br1m_ref, Wr2m_ref, br2m_ref,
           Wr1v_ref, br1v_ref, Wr2v_ref, br2v_ref, zm_ref, zv_ref, h_a, h_b,
           *, n_iter, n_b, n_tiles, n_nodes, tile):
    it = pl.program_id(0)
    i = pl.program_id(1)

    @pl.when((it == 0) & (i == 0))
    def _prologue():
        h_a[...] = h1_ref[...]

    def _step(src, dst):
        for bb in range(n_b):
            p_t = p_ref[bb]
            h8 = src[bb].astype(jnp.float8_e4m3fn)
            m = _mm(p_t, h8) * (1.0 / n_nodes)
            h_t = src[bb, pl.ds(i * tile, tile)]
            h_new = _mlp_update(h_t, m, Wm1h_ref, Wm1m_ref, bm1_ref,
                                Wm2_ref, bm2_ref, Wm3_ref, bm3_ref)
            dst[bb, pl.ds(i * tile, tile)] = h_new

        @pl.when((it == n_iter - 1) & (i == n_tiles - 1))
        def _epilogue():
            pools = [jnp.sum(dst[bb], axis=0, keepdims=True) * (1.0 / n_nodes)
                     for bb in range(n_b)]
            h_pool = jnp.concatenate(pools, axis=0)
            hm = jnp.maximum(_mm(h_pool, Wr1m_ref[...]) + br1m_ref[...], 0.0)
            zm_ref[...] = _mm(hm, Wr2m_ref[...]) + br2m_ref[...]
            hv = jnp.maximum(_mm(h_pool, Wr1v_ref[...]) + br1v_ref[...], 0.0)
            zv_ref[...] = _mm(hv, Wr2v_ref[...]) + br2v_ref[...]

    @pl.when(it % 2 == 0)
    def _even():
        _step(h_a, h_b)

    @pl.when(it % 2 == 1)
    def _odd():
        _step(h_b, h_a)


@jax.jit
def kernel(x, adj, W_in, b_in, Wm1, bm1, Wm2, bm2, Wm3, bm3,
           Wr1m, br1m, Wr2m, br2m, Wr1v, br1v, Wr2v, br2v):
    B, N, D_IN = x.shape
    D_H = W_in.shape[1]
    D_Z = Wr2m.shape[1]

    Wm1h, Wm1m = Wm1[:D_H], Wm1[D_H:]
    row = lambda v: v.reshape(1, -1)

    def full2(shape):
        return pl.BlockSpec(shape, lambda *_: (0,) * len(shape))

    t1 = _TILE1
    nt1 = N // t1
    h1, P = pl.pallas_call(
        functools.partial(_pass1, n_b=B, tile=t1, n_nodes=N),
        grid=(B, nt1),
        in_specs=[
            full2((B, N, D_IN)),
            pl.BlockSpec((1, t1, N), lambda b, i: (b, i, 0)),
            full2((D_IN, D_H)), full2((1, D_H)),
            full2((D_H, Wm1h.shape[1])), full2((D_H, Wm1m.shape[1])),
            full2((1, Wm1.shape[1])),
            full2(Wm2.shape), full2((1, Wm2.shape[1])),
            full2(Wm3.shape), full2((1, Wm3.shape[1])),
        ],
        out_specs=[pl.BlockSpec((1, t1, D_H), lambda b, i: (b, i, 0)),
                   pl.BlockSpec((1, t1, N), lambda b, i: (b, i, 0))],
        out_shape=[jax.ShapeDtypeStruct((B, N, D_H), jnp.float32),
                   jax.ShapeDtypeStruct((B, N, N), jnp.float8_e4m3fn)],
        scratch_shapes=[pltpu.VMEM((B, N, D_H), jnp.float32)],
        compiler_params=pltpu.CompilerParams(
            dimension_semantics=("arbitrary", "arbitrary")),
    )(x, adj, W_in, row(b_in), Wm1h, Wm1m, row(bm1), Wm2, row(bm2),
      Wm3, row(bm3))

    t2 = _TILE2
    nt2 = N // t2
    zm, zv = pl.pallas_call(
        functools.partial(_pass2, n_iter=_GNN_ITER - 1, n_b=B, n_tiles=nt2,
                          n_nodes=N, tile=t2),
        grid=(_GNN_ITER - 1, nt2),
        in_specs=[
            pl.BlockSpec((B, t2, N), lambda it, i: (0, i, 0)),
            full2((B, N, D_H)),
            full2((D_H, Wm1h.shape[1])), full2((D_H, Wm1m.shape[1])),
            full2((1, Wm1.shape[1])),
            full2(Wm2.shape), full2((1, Wm2.shape[1])),
            full2(Wm3.shape), full2((1, Wm3.shape[1])),
            full2(Wr1m.shape), full2((1, Wr1m.shape[1])),
            full2(Wr2m.shape), full2((1, Wr2m.shape[1])),
            full2(Wr1v.shape), full2((1, Wr1v.shape[1])),
            full2(Wr2v.shape), full2((1, Wr2v.shape[1])),
        ],
        out_specs=[pl.BlockSpec((B, D_Z), lambda it, i: (0, 0))] * 2,
        out_shape=[jax.ShapeDtypeStruct((B, D_Z), jnp.float32)] * 2,
        scratch_shapes=[pltpu.VMEM((B, N, D_H), jnp.float32),
                        pltpu.VMEM((B, N, D_H), jnp.float32)],
        compiler_params=pltpu.CompilerParams(
            dimension_semantics=("arbitrary", "arbitrary")),
    )(P, h1, Wm1h, Wm1m, row(bm1), Wm2, row(bm2), Wm3, row(bm3),
      Wr1m, row(br1m), Wr2m, row(br2m), Wr1v, row(br1v), Wr2v, row(br2v))
    return zm, zv


# speedup vs baseline: 1.0981x; 1.0981x over previous
import functools

import jax
import jax.numpy as jnp
from jax.experimental import pallas as pl
from jax.experimental.pallas import tpu as pltpu

_GNN_ITER = 3
_TILE1 = 1024
_TILE2 = 1024


def _mm(a, b):
    return jax.lax.dot_general(a, b, (((1,), (0,)), ((), ())),
                               preferred_element_type=jnp.float32)


def _mlp_update(h_t, m, Wm1h_ref, Wm1m_ref, bm1_ref, Wm2_ref, bm2_ref,
                Wm3_ref, bm3_ref):
    u = jnp.maximum(_mm(h_t, Wm1h_ref[...]) + _mm(m, Wm1m_ref[...])
                    + bm1_ref[...], 0.0)
    u = jnp.maximum(_mm(u, Wm2_ref[...]) + bm2_ref[...], 0.0)
    u = _mm(u, Wm3_ref[...]) + bm3_ref[...]
    return h_t + u


def _pass1(x_ref, adj_ref, W_in_ref, b_in_ref, Wm1h_ref, Wm1m_ref, bm1_ref,
           Wm2_ref, bm2_ref, Wm3_ref, bm3_ref, h1_ref, p_ref, h0,
           *, n_b, tile, n_nodes):
    b = pl.program_id(0)
    i = pl.program_id(1)

    @pl.when((b == 0) & (i == 0))
    def _prologue():
        for bb in range(n_b):
            h0[bb] = jnp.tanh(_mm(x_ref[bb], W_in_ref[...]) + b_in_ref[...])

    adj_t = adj_ref[0]
    deg = jnp.sum(adj_t, axis=1, keepdims=True)
    rdeg = 1.0 / jnp.maximum(deg, 1.0)
    p_ref[0] = (adj_t * (float(n_nodes) * rdeg)).astype(jnp.float8_e4m3fn)
    m = _mm(adj_t, h0[b]) * rdeg
    h_t = h0[b, pl.ds(i * tile, tile)]
    h1_ref[0] = _mlp_update(h_t, m, Wm1h_ref, Wm1m_ref, bm1_ref,
                            Wm2_ref, bm2_ref, Wm3_ref, bm3_ref)


def _pass2(p_ref, h1_ref, Wm1h_ref, Wm1m_ref, bm1_ref, Wm2_ref, bm2_ref,
           Wm3_ref, bm3_ref, Wr1m_ref, br1m_ref, Wr2m_ref, br2m_ref,
           Wr1v_ref, br1v_ref, Wr2v_ref, br2v_ref, zm_ref, zv_ref,
           h_a, h_b, h8_a, h8_b, m_buf,
           *, n_iter, n_b, n_tiles, n_nodes, tile):
    it = pl.program_id(0)
    b = pl.program_id(1)
    i = pl.program_id(2)

    @pl.when((it == 0) & (b == 0) & (i == 0))
    def _prologue():
        h_a[...] = h1_ref[...]
        h8_a[...] = h1_ref[...].astype(jnp.float8_e4m3fn)

    def _step(src, dst, src8, dst8):
        m_cur = _mm(p_ref[0], src8[b]) * (1.0 / n_nodes)
        m_buf[i % 2] = m_cur

        def _update(j, m):
            h_t = src[b, pl.ds(j * tile, tile)]
            h_new = _mlp_update(h_t, m, Wm1h_ref, Wm1m_ref, bm1_ref,
                                Wm2_ref, bm2_ref, Wm3_ref, bm3_ref)
            dst[b, pl.ds(j * tile, tile)] = h_new
            dst8[b, pl.ds(j * tile, tile)] = h_new.astype(jnp.float8_e4m3fn)

        @pl.when(i > 0)
        def _deferred():
            _update(i - 1, m_buf[(i - 1) % 2])

        @pl.when(i == n_tiles - 1)
        def _flush():
            _update(i, m_cur)

        @pl.when((it == n_iter - 1) & (b == n_b - 1) & (i == n_tiles - 1))
        def _epilogue():
            pools = [jnp.sum(dst[bb], axis=0, keepdims=True) * (1.0 / n_nodes)
                     for bb in range(n_b)]
            h_pool = jnp.concatenate(pools, axis=0)
            hm = jnp.maximum(_mm(h_pool, Wr1m_ref[...]) + br1m_ref[...], 0.0)
            zm_ref[...] = _mm(hm, Wr2m_ref[...]) + br2m_ref[...]
            hv = jnp.maximum(_mm(h_pool, Wr1v_ref[...]) + br1v_ref[...], 0.0)
            zv_ref[...] = _mm(hv, Wr2v_ref[...]) + br2v_ref[...]

    @pl.when(it % 2 == 0)
    def _even():
        _step(h_a, h_b, h8_a, h8_b)

    @pl.when(it % 2 == 1)
    def _odd():
        _step(h_b, h_a, h8_b, h8_a)


@jax.jit
def kernel(x, adj, W_in, b_in, Wm1, bm1, Wm2, bm2, Wm3, bm3,
           Wr1m, br1m, Wr2m, br2m, Wr1v, br1v, Wr2v, br2v):
    B, N, D_IN = x.shape
    D_H = W_in.shape[1]
    D_Z = Wr2m.shape[1]

    Wm1h, Wm1m = Wm1[:D_H], Wm1[D_H:]
    row = lambda v: v.reshape(1, -1)

    def full2(shape):
        return pl.BlockSpec(shape, lambda *_: (0,) * len(shape))

    t1 = _TILE1
    nt1 = N // t1
    h1, P = pl.pallas_call(
        functools.partial(_pass1, n_b=B, tile=t1, n_nodes=N),
        grid=(B, nt1),
        in_specs=[
            full2((B, N, D_IN)),
            pl.BlockSpec((1, t1, N), lambda b, i: (b, i, 0)),
            full2((D_IN, D_H)), full2((1, D_H)),
            full2((D_H, Wm1h.shape[1])), full2((D_H, Wm1m.shape[1])),
            full2((1, Wm1.shape[1])),
            full2(Wm2.shape), full2((1, Wm2.shape[1])),
            full2(Wm3.shape), full2((1, Wm3.shape[1])),
        ],
        out_specs=[pl.BlockSpec((1, t1, D_H), lambda b, i: (b, i, 0)),
                   pl.BlockSpec((1, t1, N), lambda b, i: (b, i, 0))],
        out_shape=[jax.ShapeDtypeStruct((B, N, D_H), jnp.float32),
                   jax.ShapeDtypeStruct((B, N, N), jnp.float8_e4m3fn)],
        scratch_shapes=[pltpu.VMEM((B, N, D_H), jnp.float32)],
        compiler_params=pltpu.CompilerParams(
            dimension_semantics=("arbitrary", "arbitrary")),
    )(x, adj, W_in, row(b_in), Wm1h, Wm1m, row(bm1), Wm2, row(bm2),
      Wm3, row(bm3))

    t2 = _TILE2
    nt2 = N // t2
    zm, zv = pl.pallas_call(
        functools.partial(_pass2, n_iter=_GNN_ITER - 1, n_b=B, n_tiles=nt2,
                          n_nodes=N, tile=t2),
        grid=(_GNN_ITER - 1, B, nt2),
        in_specs=[
            pl.BlockSpec((1, t2, N), lambda it, b, i: (b, i, 0)),
            full2((B, N, D_H)),
            full2((D_H, Wm1h.shape[1])), full2((D_H, Wm1m.shape[1])),
            full2((1, Wm1.shape[1])),
            full2(Wm2.shape), full2((1, Wm2.shape[1])),
            full2(Wm3.shape), full2((1, Wm3.shape[1])),
            full2(Wr1m.shape), full2((1, Wr1m.shape[1])),
            full2(Wr2m.shape), full2((1, Wr2m.shape[1])),
            full2(Wr1v.shape), full2((1, Wr1v.shape[1])),
            full2(Wr2v.shape), full2((1, Wr2v.shape[1])),
        ],
        out_specs=[pl.BlockSpec((B, D_Z), lambda it, b, i: (0, 0))] * 2,
        out_shape=[jax.ShapeDtypeStruct((B, D_Z), jnp.float32)] * 2,
        scratch_shapes=[pltpu.VMEM((B, N, D_H), jnp.float32),
                        pltpu.VMEM((B, N, D_H), jnp.float32),
                        pltpu.VMEM((B, N, D_H), jnp.float8_e4m3fn),
                        pltpu.VMEM((B, N, D_H), jnp.float8_e4m3fn),
                        pltpu.VMEM((2, t2, D_H), jnp.float32)],
        compiler_params=pltpu.CompilerParams(
            dimension_semantics=("arbitrary", "arbitrary", "arbitrary")),
    )(P, h1, Wm1h, Wm1m, row(bm1), Wm2, row(bm2), Wm3, row(bm3),
      Wr1m, row(br1m), Wr2m, row(br2m), Wr1v, row(br1v), Wr2v, row(br2v))
    return zm, zv


# speedup vs baseline: 1.1306x; 1.0296x over previous
import functools

import jax
import jax.numpy as jnp
from jax.experimental import pallas as pl
from jax.experimental.pallas import tpu as pltpu

_GNN_ITER = 3
_TILE1 = 1024
_TILE2 = 1024


def _mm(a, b):
    return jax.lax.dot_general(a, b, (((1,), (0,)), ((), ())),
                               preferred_element_type=jnp.float32)


def _augment(h):
    rows, d_h = h.shape
    ones = jnp.ones((rows, 1), h.dtype)
    zeros = jnp.zeros((rows, d_h - 1), h.dtype)
    return jnp.concatenate([h, ones, zeros], axis=1)


def _mlp_update(h_t, m, Wm1h_ref, Wm1m_ref, bm1_ref, Wm2_ref, bm2_ref,
                Wm3_ref, bm3_ref):
    u = jnp.maximum(_mm(h_t, Wm1h_ref[...]) + _mm(m, Wm1m_ref[...])
                    + bm1_ref[...], 0.0)
    u = jnp.maximum(_mm(u, Wm2_ref[...]) + bm2_ref[...], 0.0)
    u = _mm(u, Wm3_ref[...]) + bm3_ref[...]
    return h_t + u


def _pass1(x_ref, adj_ref, W_in_ref, b_in_ref, Wm1h_ref, Wm1m_ref, bm1_ref,
           Wm2_ref, bm2_ref, Wm3_ref, bm3_ref, h1_ref, p_ref, h0,
           *, n_b, tile, n_nodes):
    b = pl.program_id(0)
    i = pl.program_id(1)

    @pl.when((b == 0) & (i == 0))
    def _prologue():
        for bb in range(n_b):
            h = jnp.tanh(_mm(x_ref[bb], W_in_ref[...]) + b_in_ref[...])
            h0[bb] = _augment(h)

    adj_t = adj_ref[0]
    p_ref[0] = adj_t.astype(jnp.float8_e4m3fn)
    ah = _mm(adj_t, h0[b])
    d_h = ah.shape[1] // 2
    rdeg = 1.0 / jnp.maximum(ah[:, d_h:d_h + 1], 1.0)
    m = ah[:, :d_h] * rdeg
    h_t = h0[b, pl.ds(i * tile, tile), :d_h]
    h1_ref[0] = _mlp_update(h_t, m, Wm1h_ref, Wm1m_ref, bm1_ref,
                            Wm2_ref, bm2_ref, Wm3_ref, bm3_ref)


def _pass2(p_ref, h1_ref, Wm1h_ref, Wm1m_ref, bm1_ref, Wm2_ref, bm2_ref,
           Wm3_ref, bm3_ref, Wr1m_ref, br1m_ref, Wr2m_ref, br2m_ref,
           Wr1v_ref, br1v_ref, Wr2v_ref, br2v_ref, zm_ref, zv_ref,
           h_a, h_b, h8_a, h8_b, m_buf,
           *, n_iter, n_b, n_tiles, n_nodes, tile):
    it = pl.program_id(0)
    b = pl.program_id(1)
    i = pl.program_id(2)

    @pl.when((it == 0) & (b == 0) & (i == 0))
    def _prologue():
        h_a[...] = h1_ref[...]
        for bb in range(n_b):
            h8_a[bb] = _augment(h1_ref[bb].astype(jnp.float8_e4m3fn))

    def _step(src, dst, src8, dst8):
        d_h = m_buf.shape[2]
        ah = _mm(p_ref[0], src8[b])
        rdeg = 1.0 / jnp.maximum(ah[:, d_h:d_h + 1], 1.0)
        m_cur = ah[:, :d_h] * rdeg
        m_buf[i % 2] = m_cur

        def _update(j, m):
            h_t = src[b, pl.ds(j * tile, tile)]
            h_new = _mlp_update(h_t, m, Wm1h_ref, Wm1m_ref, bm1_ref,
                                Wm2_ref, bm2_ref, Wm3_ref, bm3_ref)
            dst[b, pl.ds(j * tile, tile)] = h_new
            dst8[b, pl.ds(j * tile, tile)] = _augment(
                h_new.astype(jnp.float8_e4m3fn))

        @pl.when(i > 0)
        def _deferred():
            _update(i - 1, m_buf[(i - 1) % 2])

        @pl.when(i == n_tiles - 1)
        def _flush():
            _update(i, m_cur)

        @pl.when((it == n_iter - 1) & (b == n_b - 1) & (i == n_tiles - 1))
        def _epilogue():
            pools = [jnp.sum(dst[bb], axis=0, keepdims=True) * (1.0 / n_nodes)
                     for bb in range(n_b)]
            h_pool = jnp.concatenate(pools, axis=0)
            hm = jnp.maximum(_mm(h_pool, Wr1m_ref[...]) + br1m_ref[...], 0.0)
            zm_ref[...] = _mm(hm, Wr2m_ref[...]) + br2m_ref[...]
            hv = jnp.maximum(_mm(h_pool, Wr1v_ref[...]) + br1v_ref[...], 0.0)
            zv_ref[...] = _mm(hv, Wr2v_ref[...]) + br2v_ref[...]

    @pl.when(it % 2 == 0)
    def _even():
        _step(h_a, h_b, h8_a, h8_b)

    @pl.when(it % 2 == 1)
    def _odd():
        _step(h_b, h_a, h8_b, h8_a)


@jax.jit
def kernel(x, adj, W_in, b_in, Wm1, bm1, Wm2, bm2, Wm3, bm3,
           Wr1m, br1m, Wr2m, br2m, Wr1v, br1v, Wr2v, br2v):
    B, N, D_IN = x.shape
    D_H = W_in.shape[1]
    D_Z = Wr2m.shape[1]

    Wm1h, Wm1m = Wm1[:D_H], Wm1[D_H:]
    row = lambda v: v.reshape(1, -1)

    def full2(shape):
        return pl.BlockSpec(shape, lambda *_: (0,) * len(shape))

    t1 = _TILE1
    nt1 = N // t1
    h1, P = pl.pallas_call(
        functools.partial(_pass1, n_b=B, tile=t1, n_nodes=N),
        grid=(B, nt1),
        in_specs=[
            full2((B, N, D_IN)),
            pl.BlockSpec((1, t1, N), lambda b, i: (b, i, 0)),
            full2((D_IN, D_H)), full2((1, D_H)),
            full2((D_H, Wm1h.shape[1])), full2((D_H, Wm1m.shape[1])),
            full2((1, Wm1.shape[1])),
            full2(Wm2.shape), full2((1, Wm2.shape[1])),
            full2(Wm3.shape), full2((1, Wm3.shape[1])),
        ],
        out_specs=[pl.BlockSpec((1, t1, D_H), lambda b, i: (b, i, 0)),
                   pl.BlockSpec((1, t1, N), lambda b, i: (b, i, 0))],
        out_shape=[jax.ShapeDtypeStruct((B, N, D_H), jnp.float32),
                   jax.ShapeDtypeStruct((B, N, N), jnp.float8_e4m3fn)],
        scratch_shapes=[pltpu.VMEM((B, N, 2 * D_H), jnp.float32)],
        compiler_params=pltpu.CompilerParams(
            dimension_semantics=("arbitrary", "arbitrary")),
    )(x, adj, W_in, row(b_in), Wm1h, Wm1m, row(bm1), Wm2, row(bm2),
      Wm3, row(bm3))

    t2 = _TILE2
    nt2 = N // t2
    zm, zv = pl.pallas_call(
        functools.partial(_pass2, n_iter=_GNN_ITER - 1, n_b=B, n_tiles=nt2,
                          n_nodes=N, tile=t2),
        grid=(_GNN_ITER - 1, B, nt2),
        in_specs=[
            pl.BlockSpec((1, t2, N), lambda it, b, i: (b, i, 0)),
            full2((B, N, D_H)),
            full2((D_H, Wm1h.shape[1])), full2((D_H, Wm1m.shape[1])),
            full2((1, Wm1.shape[1])),
            full2(Wm2.shape), full2((1, Wm2.shape[1])),
            full2(Wm3.shape), full2((1, Wm3.shape[1])),
            full2(Wr1m.shape), full2((1, Wr1m.shape[1])),
            full2(Wr2m.shape), full2((1, Wr2m.shape[1])),
            full2(Wr1v.shape), full2((1, Wr1v.shape[1])),
            full2(Wr2v.shape), full2((1, Wr2v.shape[1])),
        ],
        out_specs=[pl.BlockSpec((B, D_Z), lambda it, b, i: (0, 0))] * 2,
        out_shape=[jax.ShapeDtypeStruct((B, D_Z), jnp.float32)] * 2,
        scratch_shapes=[pltpu.VMEM((B, N, D_H), jnp.float32),
                        pltpu.VMEM((B, N, D_H), jnp.float32),
                        pltpu.VMEM((B, N, 2 * D_H), jnp.float8_e4m3fn),
                        pltpu.VMEM((B, N, 2 * D_H), jnp.float8_e4m3fn),
                        pltpu.VMEM((2, t2, D_H), jnp.float32)],
        compiler_params=pltpu.CompilerParams(
            dimension_semantics=("arbitrary", "arbitrary", "arbitrary")),
    )(P, h1, Wm1h, Wm1m, row(bm1), Wm2, row(bm2), Wm3, row(bm3),
      Wr1m, row(br1m), Wr2m, row(br2m), Wr1v, row(br1v), Wr2v, row(br2v))
    return zm, zv


# speedup vs baseline: 1.1610x; 1.0269x over previous
import functools

import jax
import jax.numpy as jnp
from jax.experimental import pallas as pl
from jax.experimental.pallas import tpu as pltpu

_GNN_ITER = 3
_TILE1 = 1024
_TILE2 = 2048


def _mm(a, b):
    return jax.lax.dot_general(a, b, (((1,), (0,)), ((), ())),
                               preferred_element_type=jnp.float32)


def _augment(h):
    rows, d_h = h.shape
    ones = jnp.ones((rows, 1), h.dtype)
    zeros = jnp.zeros((rows, d_h - 1), h.dtype)
    return jnp.concatenate([h, ones, zeros], axis=1)


def _mlp_update(h_t, m, Wm1h_ref, Wm1m_ref, bm1_ref, Wm2_ref, bm2_ref,
                Wm3_ref, bm3_ref):
    u = jnp.maximum(_mm(h_t, Wm1h_ref[...]) + _mm(m, Wm1m_ref[...])
                    + bm1_ref[...], 0.0)
    u = jnp.maximum(_mm(u, Wm2_ref[...]) + bm2_ref[...], 0.0)
    u = _mm(u, Wm3_ref[...]) + bm3_ref[...]
    return h_t + u


def _pass1(x_ref, adj_ref, W_in_ref, b_in_ref, Wm1h_ref, Wm1m_ref, bm1_ref,
           Wm2_ref, bm2_ref, Wm3_ref, bm3_ref, h1_ref, p_ref, h0,
           *, n_b, tile, n_nodes):
    b = pl.program_id(0)
    i = pl.program_id(1)

    @pl.when((b == 0) & (i == 0))
    def _prologue():
        for bb in range(n_b):
            h = jnp.tanh(_mm(x_ref[bb], W_in_ref[...]) + b_in_ref[...])
            h0[bb] = _augment(h)

    adj_t = adj_ref[0]
    p_ref[0] = adj_t.astype(jnp.float8_e4m3fn)
    ah = _mm(adj_t, h0[b])
    d_h = ah.shape[1] // 2
    rdeg = 1.0 / jnp.maximum(ah[:, d_h:d_h + 1], 1.0)
    m = ah[:, :d_h] * rdeg
    h_t = h0[b, pl.ds(i * tile, tile), :d_h]
    h1_ref[0] = _mlp_update(h_t, m, Wm1h_ref, Wm1m_ref, bm1_ref,
                            Wm2_ref, bm2_ref, Wm3_ref, bm3_ref)


def _pass2(p_ref, h1_ref, Wm1h_ref, Wm1m_ref, bm1_ref, Wm2_ref, bm2_ref,
           Wm3_ref, bm3_ref, Wr1m_ref, br1m_ref, Wr2m_ref, br2m_ref,
           Wr1v_ref, br1v_ref, Wr2v_ref, br2v_ref, zm_ref, zv_ref,
           h_a, h_b, h8_a, h8_b, m_buf,
           *, n_iter, n_b, n_tiles, n_nodes, tile):
    it = pl.program_id(0)
    b = pl.program_id(1)
    i = pl.program_id(2)

    @pl.when((it == 0) & (b == 0) & (i == 0))
    def _prologue():
        h_a[...] = h1_ref[...]
        for bb in range(n_b):
            h8_a[bb] = _augment(h1_ref[bb].astype(jnp.float8_e4m3fn))

    def _step(src, dst, src8, dst8):
        d_h = m_buf.shape[2]
        ah = _mm(p_ref[0], src8[b])
        rdeg = 1.0 / jnp.maximum(ah[:, d_h:d_h + 1], 1.0)
        m_cur = ah[:, :d_h] * rdeg
        m_buf[i % 2] = m_cur

        def _update(j, m):
            h_t = src[b, pl.ds(j * tile, tile)]
            h_new = _mlp_update(h_t, m, Wm1h_ref, Wm1m_ref, bm1_ref,
                                Wm2_ref, bm2_ref, Wm3_ref, bm3_ref)
            dst[b, pl.ds(j * tile, tile)] = h_new
            dst8[b, pl.ds(j * tile, tile)] = _augment(
                h_new.astype(jnp.float8_e4m3fn))

        @pl.when(i > 0)
        def _deferred():
            _update(i - 1, m_buf[(i - 1) % 2])

        @pl.when(i == n_tiles - 1)
        def _flush():
            _update(i, m_cur)

        @pl.when((it == n_iter - 1) & (b == n_b - 1) & (i == n_tiles - 1))
        def _epilogue():
            pools = [jnp.sum(dst[bb], axis=0, keepdims=True) * (1.0 / n_nodes)
                     for bb in range(n_b)]
            h_pool = jnp.concatenate(pools, axis=0)
            hm = jnp.maximum(_mm(h_pool, Wr1m_ref[...]) + br1m_ref[...], 0.0)
            zm_ref[...] = _mm(hm, Wr2m_ref[...]) + br2m_ref[...]
            hv = jnp.maximum(_mm(h_pool, Wr1v_ref[...]) + br1v_ref[...], 0.0)
            zv_ref[...] = _mm(hv, Wr2v_ref[...]) + br2v_ref[...]

    @pl.when(it % 2 == 0)
    def _even():
        _step(h_a, h_b, h8_a, h8_b)

    @pl.when(it % 2 == 1)
    def _odd():
        _step(h_b, h_a, h8_b, h8_a)


@jax.jit
def kernel(x, adj, W_in, b_in, Wm1, bm1, Wm2, bm2, Wm3, bm3,
           Wr1m, br1m, Wr2m, br2m, Wr1v, br1v, Wr2v, br2v):
    B, N, D_IN = x.shape
    D_H = W_in.shape[1]
    D_Z = Wr2m.shape[1]

    Wm1h, Wm1m = Wm1[:D_H], Wm1[D_H:]
    row = lambda v: v.reshape(1, -1)

    def full2(shape):
        return pl.BlockSpec(shape, lambda *_: (0,) * len(shape))

    t1 = _TILE1
    nt1 = N // t1
    h1, P = pl.pallas_call(
        functools.partial(_pass1, n_b=B, tile=t1, n_nodes=N),
        grid=(B, nt1),
        in_specs=[
            full2((B, N, D_IN)),
            pl.BlockSpec((1, t1, N), lambda b, i: (b, i, 0)),
            full2((D_IN, D_H)), full2((1, D_H)),
            full2((D_H, Wm1h.shape[1])), full2((D_H, Wm1m.shape[1])),
            full2((1, Wm1.shape[1])),
            full2(Wm2.shape), full2((1, Wm2.shape[1])),
            full2(Wm3.shape), full2((1, Wm3.shape[1])),
        ],
        out_specs=[pl.BlockSpec((1, t1, D_H), lambda b, i: (b, i, 0)),
                   pl.BlockSpec((1, t1, N), lambda b, i: (b, i, 0))],
        out_shape=[jax.ShapeDtypeStruct((B, N, D_H), jnp.float32),
                   jax.ShapeDtypeStruct((B, N, N), jnp.float8_e4m3fn)],
        scratch_shapes=[pltpu.VMEM((B, N, 2 * D_H), jnp.float32)],
        compiler_params=pltpu.CompilerParams(
            dimension_semantics=("arbitrary", "arbitrary")),
    )(x, adj, W_in, row(b_in), Wm1h, Wm1m, row(bm1), Wm2, row(bm2),
      Wm3, row(bm3))

    t2 = _TILE2
    nt2 = N // t2
    zm, zv = pl.pallas_call(
        functools.partial(_pass2, n_iter=_GNN_ITER - 1, n_b=B, n_tiles=nt2,
                          n_nodes=N, tile=t2),
        grid=(_GNN_ITER - 1, B, nt2),
        in_specs=[
            pl.BlockSpec((1, t2, N), lambda it, b, i: (b, i, 0)),
            full2((B, N, D_H)),
            full2((D_H, Wm1h.shape[1])), full2((D_H, Wm1m.shape[1])),
            full2((1, Wm1.shape[1])),
            full2(Wm2.shape), full2((1, Wm2.shape[1])),
            full2(Wm3.shape), full2((1, Wm3.shape[1])),
            full2(Wr1m.shape), full2((1, Wr1m.shape[1])),
            full2(Wr2m.shape), full2((1, Wr2m.shape[1])),
            full2(Wr1v.shape), full2((1, Wr1v.shape[1])),
            full2(Wr2v.shape), full2((1, Wr2v.shape[1])),
        ],
        out_specs=[pl.BlockSpec((B, D_Z), lambda it, b, i: (0, 0))] * 2,
        out_shape=[jax.ShapeDtypeStruct((B, D_Z), jnp.float32)] * 2,
        scratch_shapes=[pltpu.VMEM((B, N, D_H), jnp.float32),
                        pltpu.VMEM((B, N, D_H), jnp.float32),
                        pltpu.VMEM((B, N, 2 * D_H), jnp.float8_e4m3fn),
                        pltpu.VMEM((B, N, 2 * D_H), jnp.float8_e4m3fn),
                        pltpu.VMEM((2, t2, D_H), jnp.float32)],
        compiler_params=pltpu.CompilerParams(
            dimension_semantics=("arbitrary", "arbitrary", "arbitrary")),
    )(P, h1, Wm1h, Wm1m, row(bm1), Wm2, row(bm2), Wm3, row(bm3),
      Wr1m, row(br1m), Wr2m, row(br2m), Wr1v, row(br1v), Wr2v, row(br2v))
    return zm, zv


# speedup vs baseline: 1.1636x; 1.0022x over previous
import functools

import jax
import jax.numpy as jnp
from jax.experimental import pallas as pl
from jax.experimental.pallas import tpu as pltpu

_GNN_ITER = 3
_TILE1 = 1024
_TILE2 = 2048


def _mm(a, b):
    return jax.lax.dot_general(a, b, (((1,), (0,)), ((), ())),
                               preferred_element_type=jnp.float32)


def _augment(h):
    rows, d_h = h.shape
    ones = jnp.ones((rows, 1), h.dtype)
    zeros = jnp.zeros((rows, d_h - 1), h.dtype)
    return jnp.concatenate([h, ones, zeros], axis=1)


def _mlp_update(h_t, m, Wm1h_ref, Wm1m_ref, bm1_ref, Wm2_ref, bm2_ref,
                Wm3_ref, bm3_ref):
    u = jnp.maximum(_mm(h_t, Wm1h_ref[...]) + _mm(m, Wm1m_ref[...])
                    + bm1_ref[...], 0.0)
    u = jnp.maximum(_mm(u, Wm2_ref[...]) + bm2_ref[...], 0.0)
    u = _mm(u, Wm3_ref[...]) + bm3_ref[...]
    return h_t + u


def _pass1(x_ref, adj_ref, W_in_ref, b_in_ref, Wm1h_ref, Wm1m_ref, bm1_ref,
           Wm2_ref, bm2_ref, Wm3_ref, bm3_ref, h1_ref, p_ref, h0,
           *, n_b, tile):
    b = pl.program_id(0)
    i = pl.program_id(1)

    @pl.when((b == 0) & (i == 0))
    def _prologue():
        for bb in range(n_b):
            h = jnp.tanh(_mm(x_ref[bb], W_in_ref[...]) + b_in_ref[...])
            h0[bb] = _augment(h)

    adj_t = adj_ref[0]
    p_ref[0] = adj_t.astype(jnp.float8_e4m3fn)
    ah = _mm(adj_t, h0[b])
    d_h = ah.shape[1] // 2
    rdeg = 1.0 / jnp.maximum(ah[:, d_h:d_h + 1], 1.0)
    m = ah[:, :d_h] * rdeg
    h_t = h0[b, pl.ds(i * tile, tile), :d_h]
    h1_ref[0] = _mlp_update(h_t, m, Wm1h_ref, Wm1m_ref, bm1_ref,
                            Wm2_ref, bm2_ref, Wm3_ref, bm3_ref)


def _pass2(p_ref, h1_ref, Wm1h_ref, Wm1m_ref, bm1_ref, Wm2_ref, bm2_ref,
           Wm3_ref, bm3_ref, Wr1m_ref, br1m_ref, Wr2m_ref, br2m_ref,
           Wr1v_ref, br1v_ref, Wr2v_ref, br2v_ref, zm_ref, zv_ref,
           h_a, h_b, h8_a, h8_b, m_buf,
           *, n_iter, n_b, n_tiles, n_nodes, tile):
    it = pl.program_id(0)
    b = pl.program_id(1)
    i = pl.program_id(2)

    @pl.when((it == 0) & (b == 0) & (i == 0))
    def _prologue():
        h_a[...] = h1_ref[...]
        for bb in range(n_b):
            h8_a[bb] = _augment(h1_ref[bb].astype(jnp.float8_e4m3fn))

    def _step(src, dst, src8, dst8):
        d_h = m_buf.shape[2]
        ah = _mm(p_ref[0], src8[b])
        rdeg = 1.0 / jnp.maximum(ah[:, d_h:d_h + 1], 1.0)
        m_cur = ah[:, :d_h] * rdeg
        m_buf[i % 2] = m_cur

        def _update(j, m):
            h_t = src[b, pl.ds(j * tile, tile)]
            h_new = _mlp_update(h_t, m, Wm1h_ref, Wm1m_ref, bm1_ref,
                                Wm2_ref, bm2_ref, Wm3_ref, bm3_ref)
            dst[b, pl.ds(j * tile, tile)] = h_new
            dst8[b, pl.ds(j * tile, tile)] = _augment(
                h_new.astype(jnp.float8_e4m3fn))

        @pl.when(i > 0)
        def _deferred():
            _update(i - 1, m_buf[(i - 1) % 2])

        @pl.when(i == n_tiles - 1)
        def _flush():
            _update(i, m_cur)

        @pl.when((it == n_iter - 1) & (b == n_b - 1) & (i == n_tiles - 1))
        def _epilogue():
            pools = [jnp.sum(dst[bb], axis=0, keepdims=True) * (1.0 / n_nodes)
                     for bb in range(n_b)]
            h_pool = jnp.concatenate(pools, axis=0)
            hm = jnp.maximum(_mm(h_pool, Wr1m_ref[...]) + br1m_ref[...], 0.0)
            zm_ref[...] = _mm(hm, Wr2m_ref[...]) + br2m_ref[...]
            hv = jnp.maximum(_mm(h_pool, Wr1v_ref[...]) + br1v_ref[...], 0.0)
            zv_ref[...] = _mm(hv, Wr2v_ref[...]) + br2v_ref[...]

    @pl.when(it % 2 == 0)
    def _even():
        _step(h_a, h_b, h8_a, h8_b)

    @pl.when(it % 2 == 1)
    def _odd():
        _step(h_b, h_a, h8_b, h8_a)


@jax.jit
def kernel(x, adj, W_in, b_in, Wm1, bm1, Wm2, bm2, Wm3, bm3,
           Wr1m, br1m, Wr2m, br2m, Wr1v, br1v, Wr2v, br2v):
    B, N, D_IN = x.shape
    D_H = W_in.shape[1]
    D_Z = Wr2m.shape[1]

    Wm1h, Wm1m = Wm1[:D_H], Wm1[D_H:]
    row = lambda v: v.reshape(1, -1)

    def full2(shape):
        return pl.BlockSpec(shape, lambda *_: (0,) * len(shape))

    t1 = _TILE1
    nt1 = N // t1
    h1, P = pl.pallas_call(
        functools.partial(_pass1, n_b=B, tile=t1),
        grid=(B, nt1),
        in_specs=[
            full2((B, N, D_IN)),
            pl.BlockSpec((1, t1, N), lambda b, i: (b, i, 0)),
            full2((D_IN, D_H)), full2((1, D_H)),
            full2((D_H, Wm1h.shape[1])), full2((D_H, Wm1m.shape[1])),
            full2((1, Wm1.shape[1])),
            full2(Wm2.shape), full2((1, Wm2.shape[1])),
            full2(Wm3.shape), full2((1, Wm3.shape[1])),
        ],
        out_specs=[pl.BlockSpec((1, t1, D_H), lambda b, i: (b, i, 0)),
                   pl.BlockSpec((1, t1, N), lambda b, i: (b, i, 0))],
        out_shape=[jax.ShapeDtypeStruct((B, N, D_H), jnp.float32),
                   jax.ShapeDtypeStruct((B, N, N), jnp.float8_e4m3fn)],
        scratch_shapes=[pltpu.VMEM((B, N, 2 * D_H), jnp.float32)],
        compiler_params=pltpu.CompilerParams(
            dimension_semantics=("arbitrary", "arbitrary")),
    )(x, adj, W_in, row(b_in), Wm1h, Wm1m, row(bm1), Wm2, row(bm2),
      Wm3, row(bm3))

    t2 = _TILE2
    nt2 = N // t2
    zm, zv = pl.pallas_call(
        functools.partial(_pass2, n_iter=_GNN_ITER - 1, n_b=B, n_tiles=nt2,
                          n_nodes=N, tile=t2),
        grid=(_GNN_ITER - 1, B, nt2),
        in_specs=[
            pl.BlockSpec((1, t2, N), lambda it, b, i: (b, i, 0)),
            full2((B, N, D_H)),
            full2((D_H, Wm1h.shape[1])), full2((D_H, Wm1m.shape[1])),
            full2((1, Wm1.shape[1])),
            full2(Wm2.shape), full2((1, Wm2.shape[1])),
            full2(Wm3.shape), full2((1, Wm3.shape[1])),
            full2(Wr1m.shape), full2((1, Wr1m.shape[1])),
            full2(Wr2m.shape), full2((1, Wr2m.shape[1])),
            full2(Wr1v.shape), full2((1, Wr1v.shape[1])),
            full2(Wr2v.shape), full2((1, Wr2v.shape[1])),
        ],
        out_specs=[pl.BlockSpec((B, D_Z), lambda it, b, i: (0, 0))] * 2,
        out_shape=[jax.ShapeDtypeStruct((B, D_Z), jnp.float32)] * 2,
        scratch_shapes=[pltpu.VMEM((B, N, D_H), jnp.float32),
                        pltpu.VMEM((B, N, D_H), jnp.float32),
                        pltpu.VMEM((B, N, 2 * D_H), jnp.float8_e4m3fn),
                        pltpu.VMEM((B, N, 2 * D_H), jnp.float8_e4m3fn),
                        pltpu.VMEM((2, t2, D_H), jnp.float32)],
        compiler_params=pltpu.CompilerParams(
            dimension_semantics=("arbitrary", "arbitrary", "arbitrary")),
    )(P, h1, Wm1h, Wm1m, row(bm1), Wm2, row(bm2), Wm3, row(bm3),
      Wr1m, row(br1m), Wr2m, row(br2m), Wr1v, row(br1v), Wr2v, row(br2v))
    return zm, zv
